# baseline (speedup 1.0000x reference)
import functools

import jax
import jax.numpy as jnp
from jax import lax
from jax.experimental import pallas as pl
from jax.experimental.pallas import tpu as pltpu
from jax.experimental.pallas import tpu_sc as plsc

_INPUT_DIM = 3
_PARTITION_NUM = 100
_OUTPUT_DIM = 16
_B = 16384

_info = plsc.get_sparse_core_info()
_NC, _NS, _L = _info.num_cores, _info.num_subcores, _info.num_lanes
_NW = _NC * _NS
_B_PER_W = _B // _NW
_CHUNK = 128
_NCHUNK = _B_PER_W // _CHUNK


def _body(c0_hbm, c1_hbm, c2_hbm, table_hbm, out_hbm, c0v, c1v, c2v,
          i0, i1, i2, i3, r0, r1, r2, r3, sem):
    col_hbm = (c0_hbm, c1_hbm, c2_hbm)
    col_bufs = (c0v, c1v, c2v)
    idx_bufs = (i0, i1, i2, i3)
    row_bufs = (r0, r1, r2, r3)
    wid = lax.axis_index("s") * _NC + lax.axis_index("c")
    base = wid * _B_PER_W

    for d in range(_INPUT_DIM):
        pltpu.sync_copy(col_hbm[d].at[pl.ds(base, _B_PER_W)], col_bufs[d])

    copies = []
    for j in range(_NCHUNK):
        for t in range(_CHUNK // _L):
            g = j * _CHUNK + t * _L
            digits = []
            for d in range(_INPUT_DIM):
                x = col_bufs[d][pl.ds(g, _L)]
                x = jnp.maximum(x, 0.0)
                s = (x * jnp.float32(_PARTITION_NUM)).astype(jnp.int32)
                digits.append(jnp.minimum(s, _PARTITION_NUM - 1))
            idx = digits[0] + digits[1] * _PARTITION_NUM \
                + digits[2] * (_PARTITION_NUM * _PARTITION_NUM)
            idx_bufs[j][pl.ds(t * _L, _L)] = idx
        copies.append(pltpu.async_copy(table_hbm.at[idx_bufs[j]],
                                       row_bufs[j], sem))

    for j in range(_NCHUNK):
        copies[j].wait()
        pltpu.sync_copy(row_bufs[j],
                        out_hbm.at[pl.ds(base + j * _CHUNK, _CHUNK)])


@jax.jit
def kernel(inputs, table):
    mesh = plsc.VectorSubcoreMesh(core_axis_name="c", subcore_axis_name="s")
    fn = pl.kernel(
        _body,
        mesh=mesh,
        compiler_params=pltpu.CompilerParams(use_tc_tiling_on_sc=False),
        out_type=jax.ShapeDtypeStruct((_B, _OUTPUT_DIM), jnp.float32),
        scratch_types=[
            pltpu.VMEM((_B_PER_W,), jnp.float32),
            pltpu.VMEM((_B_PER_W,), jnp.float32),
            pltpu.VMEM((_B_PER_W,), jnp.float32),
            pltpu.VMEM((_CHUNK,), jnp.int32),
            pltpu.VMEM((_CHUNK,), jnp.int32),
            pltpu.VMEM((_CHUNK,), jnp.int32),
            pltpu.VMEM((_CHUNK,), jnp.int32),
            pltpu.VMEM((_CHUNK, _OUTPUT_DIM), jnp.float32),
            pltpu.VMEM((_CHUNK, _OUTPUT_DIM), jnp.float32),
            pltpu.VMEM((_CHUNK, _OUTPUT_DIM), jnp.float32),
            pltpu.VMEM((_CHUNK, _OUTPUT_DIM), jnp.float32),
            pltpu.SemaphoreType.DMA,
        ],
    )
    cols = inputs.T
    return fn(cols[0], cols[1], cols[2], table)
